# baseline (speedup 1.0000x reference)
import jax
import jax.numpy as jnp
from jax.experimental import pallas as pl
from jax.experimental.pallas import tpu as pltpu

_VOCAB = 1000
_SBLK = 512


def _onehot_t_block(ids_ref, out_ref):
    ids = ids_ref[...].astype(jnp.int32)
    nxt = (ids + 1) % _VOCAB
    row = jax.lax.broadcasted_iota(jnp.int32, (_VOCAB, _SBLK), 0)
    out_ref[0] = jnp.where(row == nxt[None, :], jnp.float32(12.0), jnp.float32(0.0))


def kernel(input_ids, anchor):
    B, S = input_ids.shape
    nsb = S // _SBLK
    flat_ids = input_ids.reshape(B * S).astype(jnp.int32)
    out_t = pl.pallas_call(
        _onehot_t_block,
        grid=(B, nsb),
        in_specs=[pl.BlockSpec((_SBLK,), lambda b, j: (b * nsb + j,))],
        out_specs=pl.BlockSpec((1, _VOCAB, _SBLK), lambda b, j: (b, 0, j)),
        out_shape=jax.ShapeDtypeStruct((B, _VOCAB, S), jnp.float32),
        compiler_params=pltpu.CompilerParams(
            dimension_semantics=("parallel", "parallel"),
        ),
    )(flat_ids)
    return jnp.swapaxes(out_t, 1, 2)


# speedup vs baseline: 1.3115x; 1.3115x over previous
import jax
import jax.numpy as jnp
from jax.experimental import pallas as pl
from jax.experimental.pallas import tpu as pltpu

_VOCAB = 1000
_SBLK = 2048
_BBLK = 2


def _onehot_t_block(ids_ref, out_ref):
    row = jax.lax.broadcasted_iota(jnp.int32, (_VOCAB, _SBLK), 0)
    for i in range(_BBLK):
        ids = ids_ref[pl.ds(i * _SBLK, _SBLK)].astype(jnp.int32)
        nxt = (ids + 1) % _VOCAB
        out_ref[i] = jnp.where(
            row == nxt[None, :], jnp.float32(12.0), jnp.float32(0.0)
        )


def kernel(input_ids, anchor):
    B, S = input_ids.shape
    flat_ids = input_ids.reshape(B * S).astype(jnp.int32)
    out_t = pl.pallas_call(
        _onehot_t_block,
        grid=(B // _BBLK,),
        in_specs=[pl.BlockSpec((_BBLK * _SBLK,), lambda b: (b,))],
        out_specs=pl.BlockSpec((_BBLK, _VOCAB, _SBLK), lambda b: (b, 0, 0)),
        out_shape=jax.ShapeDtypeStruct((B, _VOCAB, S), jnp.float32),
        compiler_params=pltpu.CompilerParams(
            dimension_semantics=("parallel",),
        ),
    )(flat_ids)
    return jnp.swapaxes(out_t, 1, 2)


# speedup vs baseline: 1.3329x; 1.0163x over previous
import jax
import jax.numpy as jnp
from jax.experimental import pallas as pl
from jax.experimental.pallas import tpu as pltpu

_VOCAB = 1000
_SBLK = 2048


def _onehot_t_block(ids_ref, out_ref):
    ids = ids_ref[...].astype(jnp.int32)
    nxt = (ids + 1) % _VOCAB
    row = jax.lax.broadcasted_iota(jnp.int32, (_VOCAB, _SBLK), 0)
    out_ref[0] = jnp.where(row == nxt[None, :], jnp.float32(12.0), jnp.float32(0.0))


def kernel(input_ids, anchor):
    B, S = input_ids.shape
    nsb = S // _SBLK
    flat_ids = input_ids.reshape(B * S).astype(jnp.int32)
    out_t = pl.pallas_call(
        _onehot_t_block,
        grid=(B, nsb),
        in_specs=[pl.BlockSpec((_SBLK,), lambda b, j: (b * nsb + j,))],
        out_specs=pl.BlockSpec((1, _VOCAB, _SBLK), lambda b, j: (b, 0, j)),
        out_shape=jax.ShapeDtypeStruct((B, _VOCAB, S), jnp.float32),
        compiler_params=pltpu.CompilerParams(
            dimension_semantics=("parallel", "parallel"),
        ),
    )(flat_ids)
    return jnp.swapaxes(out_t, 1, 2)
